# baseline (speedup 1.0000x reference)
import functools

import jax
import jax.numpy as jnp
from jax import lax
from jax.experimental import pallas as pl
from jax.experimental.pallas import tpu as pltpu
from jax.experimental.pallas import tpu_sc as plsc

D_MODEL = 64
ROW2 = 2 * D_MODEL
NC, NS = 2, 16
NW = NC * NS
NBUF = 3
CHUNK = 128
LANES = 16


def _make_kernel(n_idx):
    n_chunks_w = n_idx // (NW * CHUNK)
    mesh = plsc.VectorSubcoreMesh(core_axis_name="c", subcore_axis_name="s")

    scratch = [
        pltpu.VMEM((n_chunks_w, CHUNK), jnp.int32),
        pltpu.VMEM((n_chunks_w * CHUNK,), jnp.int32),
        pltpu.VMEM((NBUF, CHUNK, ROW2), jnp.float32),
        pltpu.VMEM((NBUF, CHUNK // 2, ROW2), jnp.float32),
    ] + [pltpu.SemaphoreType.DMA] * (2 * NBUF)

    @functools.partial(
        pl.kernel,
        out_type=jax.ShapeDtypeStruct((n_idx // 2, ROW2), jnp.float32),
        mesh=mesh,
        scratch_types=scratch,
        compiler_params=pltpu.CompilerParams(use_tc_tiling_on_sc=False),
    )
    def k(x_hbm, w_hbm, out_hbm, idx_v, half_v, rows, sel, *sems):
        gsem = sems[:NBUF]
        wsem = sems[NBUF:]
        wid = lax.axis_index("s") * NC + lax.axis_index("c")
        iota = lax.iota(jnp.int32, LANES)

        pltpu.sync_copy(x_hbm.at[pl.ds(wid * n_chunks_w, n_chunks_w)], idx_v)

        @pl.loop(0, n_chunks_w * (CHUNK // LANES))
        def _(g):
            r = g // (CHUNK // LANES)
            c0 = (g % (CHUNK // LANES)) * LANES
            v = idx_v[r, pl.ds(c0, LANES)]
            half_v[pl.ds(g * LANES, LANES)] = (v & 1) * D_MODEL
            idx_v[r, pl.ds(c0, LANES)] = v >> 1

        def start_gather(c, s):
            pltpu.async_copy(w_hbm.at[idx_v.at[c]], rows.at[s], gsem[s])

        def wait_gather(s):
            pltpu.make_async_copy(w_hbm.at[idx_v.at[0]], rows.at[s],
                                  gsem[s]).wait()

        def select(c, s):
            @pl.loop(0, CHUNK // LANES)
            def _(q):
                hvec = half_v[pl.ds(c * CHUNK + q * LANES, LANES)]
                for l in range(LANES):
                    r = q * LANES + l
                    hb = hvec[l]
                    j = q * (LANES // 2) + l // 2
                    h = l & 1
                    for u in range(D_MODEL // LANES):
                        sel[s, j, pl.ds(h * D_MODEL + u * LANES, LANES)] = (
                            rows[s, r, pl.ds(hb + u * LANES, LANES)])

        def start_write(c, s):
            base = (wid * n_chunks_w + c) * (CHUNK // 2)
            dst = out_hbm.at[pl.ds(base, CHUNK // 2)]
            pltpu.async_copy(sel.at[s], dst, wsem[s])

        def wait_write(s):
            dst = out_hbm.at[pl.ds(0, CHUNK // 2)]
            pltpu.make_async_copy(sel.at[s], dst, wsem[s]).wait()

        for s in range(NBUF):
            start_gather(s, s)

        @pl.loop(0, n_chunks_w // NBUF)
        def _(t):
            co = t * NBUF
            for s in range(NBUF):
                wait_gather(s)

                @pl.when(co + s >= NBUF)
                def _():
                    wait_write(s)
                select(co + s, s)
                start_write(co + s, s)
            for s in range(NBUF):
                nxt = co + NBUF + s

                @pl.when(nxt < n_chunks_w)
                def _():
                    start_gather(nxt, s)

        for c in range((n_chunks_w // NBUF) * NBUF, n_chunks_w):
            s = c % NBUF
            wait_gather(s)
            wait_write(s)
            select(c, s)
            start_write(c, s)

        for s in range(NBUF):
            wait_write(s)

    return k


def kernel(x, w):
    B, S = x.shape
    n_idx = B * S
    x2d = x.astype(jnp.int32).reshape(n_idx // CHUNK, CHUNK)
    w2 = w.reshape(w.shape[0] // 2, ROW2)
    out = _make_kernel(n_idx)(x2d, w2)
    return out.reshape(B, S, D_MODEL)


# speedup vs baseline: 1.3322x; 1.3322x over previous
import functools

import jax
import jax.numpy as jnp
from jax import lax
from jax.experimental import pallas as pl
from jax.experimental.pallas import tpu as pltpu
from jax.experimental.pallas import tpu_sc as plsc

D_MODEL = 64
NC, NS = 2, 16
NW = NC * NS
NBUF = 3
CHUNK = 128
BAT = 4


def _make_kernel(n_idx):
    n_chunks_w = n_idx // (NW * CHUNK)
    n_blocks_w = n_chunks_w // BAT
    mesh = plsc.VectorSubcoreMesh(core_axis_name="c", subcore_axis_name="s")

    scratch = [
        pltpu.VMEM((n_chunks_w, CHUNK), jnp.int32),
        pltpu.VMEM((NBUF, BAT, CHUNK, D_MODEL), jnp.float32),
    ] + [pltpu.SemaphoreType.DMA] * (2 * NBUF)

    @functools.partial(
        pl.kernel,
        out_type=jax.ShapeDtypeStruct((n_idx // CHUNK, CHUNK, D_MODEL),
                                      jnp.float32),
        mesh=mesh,
        scratch_types=scratch,
        compiler_params=pltpu.CompilerParams(use_tc_tiling_on_sc=False),
    )
    def k(x_hbm, w_hbm, out_hbm, idx_v, blocks, *sems):
        gsem = sems[:NBUF]
        wsem = sems[NBUF:]
        wid = lax.axis_index("s") * NC + lax.axis_index("c")

        pltpu.sync_copy(x_hbm.at[pl.ds(wid * n_chunks_w, n_chunks_w)], idx_v)

        def start_gather(b, s):
            for j in range(BAT):
                pltpu.async_copy(w_hbm.at[idx_v.at[b * BAT + j]],
                                 blocks.at[s, j], gsem[s])

        def wait_gather(s):
            for j in range(BAT):
                pltpu.make_async_copy(w_hbm.at[idx_v.at[0]],
                                      blocks.at[s, j], gsem[s]).wait()

        def start_write(b, s):
            base = wid * n_chunks_w + b * BAT
            dst = out_hbm.at[pl.ds(base, BAT)]
            pltpu.async_copy(blocks.at[s], dst, wsem[s])

        def wait_write(s):
            dst = out_hbm.at[pl.ds(0, BAT)]
            pltpu.make_async_copy(blocks.at[s], dst, wsem[s]).wait()

        for s in range(NBUF):
            start_gather(s, s)

        @pl.loop(0, n_blocks_w // NBUF)
        def _(t):
            bo = t * NBUF
            for s in range(NBUF):
                wait_gather(s)
                start_write(bo + s, s)
            for s in range(NBUF):
                nxt = bo + NBUF + s

                @pl.when(nxt < n_blocks_w)
                def _():
                    wait_write(s)
                    start_gather(nxt, s)

        for b in range((n_blocks_w // NBUF) * NBUF, n_blocks_w):
            s = b % NBUF
            wait_gather(s)
            start_write(b, s)

        for s in range(NBUF):
            wait_write(s)

    return k


def kernel(x, w):
    B, S = x.shape
    n_idx = B * S
    x2d = x.astype(jnp.int32).reshape(n_idx // CHUNK, CHUNK)
    out = _make_kernel(n_idx)(x2d, w)
    return out.reshape(B, S, D_MODEL)
